# baseline (speedup 1.0000x reference)
import functools

import jax
import jax.numpy as jnp
from jax import lax
from jax.experimental import pallas as pl
from jax.experimental.pallas import tpu as pltpu
from jax.experimental.pallas import tpu_sc as plsc

_B = 16384
_D = 3
_NC, _NS, _L = 2, 16, 16
_NW = _NC * _NS
_BPW = _B // _NW
_STEPS = _BPW // _L

_U_ROWS = 1500
_I_ROWS = 2000
_IT_BASE = 4504
_CHUNK = 664
_TAB = _CHUNK * _NS

_mesh = plsc.VectorSubcoreMesh(core_axis_name="c", subcore_axis_name="s")


@functools.partial(
    pl.kernel,
    out_type=jax.ShapeDtypeStruct((_B,), jnp.float32),
    mesh=_mesh,
    compiler_params=pltpu.CompilerParams(needs_layout_passes=False,
                                         use_tc_tiling_on_sc=False,
                                         disable_bounds_checks=True,
                                         skip_device_barrier=True),
    scratch_types=[
        pltpu.VMEM((2, _BPW), jnp.int32),
        pltpu.VMEM_SHARED((_TAB,), jnp.float32),
        pltpu.VMEM((_TAB,), jnp.float32),
        pltpu.VMEM((_BPW,), jnp.float32),
        pltpu.SemaphoreType.DMA,
    ],
)
def _mf_kernel(data_hbm, tabs_hbm, out_hbm,
               idx_v, tabs_sh, tabs_v, out_v, sem):
    s = lax.axis_index("s")
    wid = s * _NC + lax.axis_index("c")
    base = wid * _BPW

    idx_cp = pltpu.async_copy(data_hbm.at[:, pl.ds(base, _BPW)], idx_v, sem)

    chunk = s * _CHUNK
    pltpu.sync_copy(tabs_hbm.at[pl.ds(chunk, _CHUNK)],
                    tabs_sh.at[pl.ds(chunk, _CHUNK)])
    plsc.subcore_barrier()
    pltpu.sync_copy(tabs_sh, tabs_v)
    idx_cp.wait()

    for step in range(_STEPS):
        off = step * _L
        iu = idx_v[0, pl.ds(off, _L)] * _D
        iv = idx_v[1, pl.ds(off, _L)] * _D + _IT_BASE
        acc = plsc.load_gather(tabs_v, [iu]) * plsc.load_gather(tabs_v, [iv])
        for d in range(1, _D):
            acc = acc + (plsc.load_gather(tabs_v, [iu + d]) *
                         plsc.load_gather(tabs_v, [iv + d]))
        out_v[pl.ds(off, _L)] = acc

    pltpu.sync_copy(out_v, out_hbm.at[pl.ds(base, _BPW)])


def kernel(data, user_factors, item_factors):
    uf = user_factors.reshape(-1)
    itf = item_factors.reshape(-1)
    tabs = jnp.concatenate([
        uf,
        jnp.zeros((_IT_BASE - _U_ROWS * _D,), jnp.float32),
        itf,
        jnp.zeros((_TAB - _IT_BASE - _I_ROWS * _D,), jnp.float32),
    ])
    return _mf_kernel(data.astype(jnp.int32), tabs)


# speedup vs baseline: 1.0039x; 1.0039x over previous
import functools

import jax
import jax.numpy as jnp
from jax import lax
from jax.experimental import pallas as pl
from jax.experimental.pallas import tpu as pltpu
from jax.experimental.pallas import tpu_sc as plsc

_B = 16384
_D = 3
_NC, _NS, _L = 2, 16, 16
_NW = _NC * _NS
_BPW = _B // _NW
_STEPS = _BPW // _L

_U_ROWS = 1500
_I_ROWS = 2000
_IT_BASE = 4504
_CHUNK = 664
_TAB = _CHUNK * _NS

_mesh = plsc.VectorSubcoreMesh(core_axis_name="c", subcore_axis_name="s")


@functools.partial(
    pl.kernel,
    out_type=jax.ShapeDtypeStruct((_B,), jnp.float32),
    mesh=_mesh,
    compiler_params=pltpu.CompilerParams(needs_layout_passes=False,
                                         use_tc_tiling_on_sc=False),
    scratch_types=[
        pltpu.VMEM((2, _BPW), jnp.int32),
        pltpu.VMEM_SHARED((_TAB,), jnp.float32),
        pltpu.VMEM((_TAB,), jnp.float32),
        pltpu.VMEM((_BPW,), jnp.float32),
        pltpu.SemaphoreType.DMA,
    ],
)
def _mf_kernel(data_hbm, tabs_hbm, out_hbm,
               idx_v, tabs_sh, tabs_v, out_v, sem):
    s = lax.axis_index("s")
    wid = s * _NC + lax.axis_index("c")
    base = wid * _BPW

    idx_cp = pltpu.async_copy(data_hbm.at[:, pl.ds(base, _BPW)], idx_v, sem)

    chunk = s * _CHUNK
    pltpu.sync_copy(tabs_hbm.at[pl.ds(chunk, _CHUNK)],
                    tabs_sh.at[pl.ds(chunk, _CHUNK)])
    plsc.subcore_barrier()
    pltpu.sync_copy(tabs_sh, tabs_v)
    idx_cp.wait()

    for step in range(_STEPS):
        off = step * _L
        iu = idx_v[0, pl.ds(off, _L)] * _D
        iv = idx_v[1, pl.ds(off, _L)] * _D + _IT_BASE
        acc = plsc.load_gather(tabs_v, [iu]) * plsc.load_gather(tabs_v, [iv])
        for d in range(1, _D):
            acc = acc + (plsc.load_gather(tabs_v, [iu + d]) *
                         plsc.load_gather(tabs_v, [iv + d]))
        out_v[pl.ds(off, _L)] = acc

    pltpu.sync_copy(out_v, out_hbm.at[pl.ds(base, _BPW)])


def kernel(data, user_factors, item_factors):
    uf = user_factors.reshape(-1)
    itf = item_factors.reshape(-1)
    tabs = jnp.concatenate([
        uf,
        jnp.zeros((_IT_BASE - _U_ROWS * _D,), jnp.float32),
        itf,
        jnp.zeros((_TAB - _IT_BASE - _I_ROWS * _D,), jnp.float32),
    ])
    return _mf_kernel(data.astype(jnp.int32), tabs)


# speedup vs baseline: 1.0181x; 1.0142x over previous
import functools

import jax
import jax.numpy as jnp
from jax import lax
from jax.experimental import pallas as pl
from jax.experimental.pallas import tpu as pltpu
from jax.experimental.pallas import tpu_sc as plsc

_B = 16384
_D = 3
_NC, _NS, _L = 2, 16, 16
_NW = _NC * _NS
_BPW = _B // _NW
_STEPS = _BPW // _L

_U_ROWS = 1500
_I_ROWS = 2000
_IT_BASE = 4504
_CHUNK = 664
_TAB = _CHUNK * _NS
_TAB_OFF = 2 * _B

_mesh = plsc.VectorSubcoreMesh(core_axis_name="c", subcore_axis_name="s")


@functools.partial(
    pl.kernel,
    out_type=jax.ShapeDtypeStruct((_B,), jnp.float32),
    mesh=_mesh,
    compiler_params=pltpu.CompilerParams(needs_layout_passes=False,
                                         use_tc_tiling_on_sc=False),
    scratch_types=[
        pltpu.VMEM((_BPW,), jnp.float32),
        pltpu.VMEM((_BPW,), jnp.float32),
        pltpu.VMEM_SHARED((_TAB,), jnp.float32),
        pltpu.VMEM((_TAB,), jnp.float32),
        pltpu.VMEM((_BPW,), jnp.float32),
        pltpu.SemaphoreType.DMA,
    ],
)
def _mf_kernel(allin_hbm, out_hbm,
               uidx_v, iidx_v, tabs_sh, tabs_v, out_v, sem):
    s = lax.axis_index("s")
    wid = s * _NC + lax.axis_index("c")
    base = wid * _BPW

    cps = [
        pltpu.async_copy(allin_hbm.at[pl.ds(base, _BPW)], uidx_v, sem),
        pltpu.async_copy(allin_hbm.at[pl.ds(_B + base, _BPW)], iidx_v, sem),
    ]

    chunk = s * _CHUNK
    pltpu.sync_copy(allin_hbm.at[pl.ds(_TAB_OFF + chunk, _CHUNK)],
                    tabs_sh.at[pl.ds(chunk, _CHUNK)])
    plsc.subcore_barrier()
    pltpu.sync_copy(tabs_sh, tabs_v)
    for cp in cps:
        cp.wait()

    for step in range(_STEPS):
        off = step * _L
        iu = plsc.bitcast(uidx_v[pl.ds(off, _L)], jnp.int32) * _D
        iv = plsc.bitcast(iidx_v[pl.ds(off, _L)], jnp.int32) * _D + _IT_BASE
        acc = plsc.load_gather(tabs_v, [iu]) * plsc.load_gather(tabs_v, [iv])
        for d in range(1, _D):
            acc = acc + (plsc.load_gather(tabs_v, [iu + d]) *
                         plsc.load_gather(tabs_v, [iv + d]))
        out_v[pl.ds(off, _L)] = acc

    pltpu.sync_copy(out_v, out_hbm.at[pl.ds(base, _BPW)])


def kernel(data, user_factors, item_factors):
    dflat = lax.bitcast_convert_type(data.astype(jnp.int32),
                                     jnp.float32).reshape(-1)
    allin = jnp.concatenate([
        dflat,
        user_factors.reshape(-1),
        jnp.zeros((_IT_BASE - _U_ROWS * _D,), jnp.float32),
        item_factors.reshape(-1),
        jnp.zeros((_TAB - _IT_BASE - _I_ROWS * _D,), jnp.float32),
    ])
    return _mf_kernel(allin)
